# Initial kernel scaffold; baseline (speedup 1.0000x reference)
import jax
import jax.numpy as jnp
from jax.experimental import pallas as pl


def kernel(x, src0, dst0, src1, dst1, n_id0, n_id1, batch_size, hist0, hist1, W_self1, W_neigh1, b1, W_self2, W_neigh2, b2):
    raise NotImplementedError("write your pallas kernel here")


# speedup vs baseline: 1.2352x; 1.2352x over previous
import functools

import jax
import jax.numpy as jnp
from jax import lax
from jax.experimental import pallas as pl
from jax.experimental.pallas import tpu as pltpu
from jax.experimental.pallas import tpu_sc as plsc

NUM_NODES = 100000
BS = 10000
N_DST0 = 20000
N_DST1 = 10000
E0 = 320000
E1 = 160000
L = 16
CH = 64
G = 64


def _make_pull(n_pull, d):
    out_rows = BS + n_pull + 1
    dummy = BS + n_pull
    n_full = n_pull // CH
    rem = n_pull % CH
    assert rem % L == 0
    ncopy = BS // CH
    copy_rem = BS % CH
    mesh = plsc.VectorSubcoreMesh(core_axis_name="c", subcore_axis_name="s")

    @functools.partial(
        pl.kernel,
        mesh=mesh,
        compiler_params=pltpu.CompilerParams(needs_layout_passes=False),
        out_type=jax.ShapeDtypeStruct((out_rows, d), jnp.float32),
        scratch_types=[
            pltpu.VMEM((NUM_NODES,), jnp.int32),
            pltpu.VMEM((BS,), jnp.int32),
            pltpu.VMEM((CH,), jnp.int32),
            pltpu.VMEM((CH,), jnp.int32),
            pltpu.VMEM((CH,), jnp.int32),
            pltpu.VMEM((CH, d), jnp.float32),
            pltpu.VMEM((CH, d), jnp.float32),
            pltpu.SemaphoreType.DMA,
            pltpu.SemaphoreType.DMA,
        ],
    )
    def k(src_hbm, nid_hbm, hist_hbm, full_hbm,
          map_v, push_v, pid_v, gid_v, did_v, buf0, buf1, sem0, sem1):
        cid = lax.axis_index("c")
        sid = lax.axis_index("s")
        wid = cid * 16 + sid
        iota = lax.iota(jnp.int32, L)

        def copy_body(kk, _):
            i = wid + 32 * kk

            @pl.when(i < ncopy)
            def _():
                pltpu.sync_copy(src_hbm.at[pl.ds(i * CH, CH)], buf0)
                pltpu.sync_copy(buf0, full_hbm.at[pl.ds(i * CH, CH)])
            return 0

        lax.fori_loop(0, (ncopy + 31) // 32, copy_body, 0)
        if copy_rem:
            @pl.when(wid == 0)
            def _():
                pltpu.sync_copy(src_hbm.at[pl.ds(ncopy * CH, copy_rem)],
                                buf0.at[pl.ds(0, copy_rem)])
                pltpu.sync_copy(buf0.at[pl.ds(0, copy_rem)],
                                full_hbm.at[pl.ds(ncopy * CH, copy_rem)])

        neg1 = jnp.full((L,), -1, jnp.int32)

        def init_body(i, _):
            base = i * (8 * L)
            for t in range(8):
                map_v[pl.ds(base + t * L, L)] = neg1
            return 0

        lax.fori_loop(0, NUM_NODES // (8 * L), init_body, 0)
        base0 = (NUM_NODES // (8 * L)) * (8 * L)
        for t in range((NUM_NODES % (8 * L)) // L):
            map_v[pl.ds(base0 + t * L, L)] = neg1
        pltpu.sync_copy(nid_hbm.at[pl.ds(0, BS)], push_v)

        def scat_body(i, _):
            nv = push_v[pl.ds(i * L, L)]
            plsc.store_scatter(map_v, [nv], iota + i * L)
            return 0

        lax.fori_loop(0, BS // L, scat_body, 0)

        def do_chunk(i):
            pltpu.sync_copy(nid_hbm.at[pl.ds(BS + i * CH, CH)], pid_v)
            cp = pltpu.async_copy(hist_hbm.at[pid_v], buf0, sem0)
            for t in range(CH // L):
                p16 = pid_v[pl.ds(t * L, L)]
                m = plsc.load_gather(map_v, [p16])
                has = m >= 0
                grow = (BS + i * CH + t * L) + iota
                gid_v[pl.ds(t * L, L)] = jnp.maximum(m, 0)
                did_v[pl.ds(t * L, L)] = jnp.where(has, grow, dummy)
            cp.wait()
            pltpu.sync_copy(buf0, full_hbm.at[pl.ds(BS + i * CH, CH)])
            pltpu.async_copy(src_hbm.at[gid_v], buf1, sem1).wait()
            pltpu.async_copy(buf1, full_hbm.at[did_v], sem1).wait()

        def pull_body(kk, _):
            i = wid + 32 * kk

            @pl.when(i < n_full)
            def _():
                do_chunk(i)
            return 0

        lax.fori_loop(0, (n_full + 31) // 32, pull_body, 0)

        if rem:
            @pl.when(wid == 1)
            def _():
                base = BS + n_full * CH
                pltpu.sync_copy(nid_hbm.at[pl.ds(base, rem)],
                                pid_v.at[pl.ds(0, rem)])
                cp = pltpu.async_copy(hist_hbm.at[pid_v.at[pl.ds(0, rem)]],
                                      buf0.at[pl.ds(0, rem)], sem0)
                for t in range(CH // L):
                    if t < rem // L:
                        p16 = pid_v[pl.ds(t * L, L)]
                        m = plsc.load_gather(map_v, [p16])
                        grow = (base + t * L) + iota
                        gid_v[pl.ds(t * L, L)] = jnp.maximum(m, 0)
                        did_v[pl.ds(t * L, L)] = jnp.where(m >= 0, grow, dummy)
                    else:
                        gid_v[pl.ds(t * L, L)] = jnp.zeros((L,), jnp.int32)
                        did_v[pl.ds(t * L, L)] = jnp.full((L,), dummy, jnp.int32)
                cp.wait()
                pltpu.sync_copy(buf0.at[pl.ds(0, rem)],
                                full_hbm.at[pl.ds(base, rem)])
                pltpu.async_copy(src_hbm.at[gid_v], buf1, sem1).wait()
                pltpu.async_copy(buf1, full_hbm.at[did_v], sem1).wait()

    return k


def _make_conv(E, n_dst, d, with_s, R, npass):
    half = n_dst // 2
    quarter = half // npass
    assert half % npass == 0
    pad = ((quarter + 1 + 63) // 64) * 64
    ept = E // 16
    nech = quarter // R
    assert quarter % R == 0
    bufrows = max(G, R)
    mesh = plsc.VectorSubcoreMesh(core_axis_name="c", subcore_axis_name="s")

    seg = 2000
    nseg = ept // seg
    assert ept % seg == 0 and seg % L == 0
    scratch = [
        pltpu.VMEM((seg,), jnp.int32),
        pltpu.VMEM((seg,), jnp.int32),
        pltpu.VMEM((seg + 128,), jnp.int32),
        pltpu.VMEM((seg + 128,), jnp.int32),
        pltpu.VMEM((bufrows, d), jnp.float32),
        pltpu.VMEM((G,), jnp.int32),
        pltpu.VMEM((G,), jnp.float32),
        pltpu.VMEM((64,), jnp.float32),
        pltpu.VMEM((64,), jnp.float32),
        pltpu.VMEM_SHARED((pad, d), jnp.float32),
        pltpu.VMEM_SHARED((pad,), jnp.float32),
        pltpu.SemaphoreType.DMA,
    ]
    if with_s:
        scratch.insert(9, pltpu.VMEM((64, d), jnp.float32))

    def body(*refs):
        if with_s:
            (tbl_hbm, src_hbm, dst_hbm, out_hbm,
             srcv, dstv, csrc, cdst, buf0, addidx, degones, zdeg, degbuf,
             sbuf, acc_sh, deg_sh, sem0) = refs
        else:
            (tbl_hbm, src_hbm, dst_hbm, out_hbm,
             srcv, dstv, csrc, cdst, buf0, addidx, degones, zdeg, degbuf,
             acc_sh, deg_sh, sem0) = refs
        cid = lax.axis_index("c")
        sid = lax.axis_index("s")
        zeros = jnp.zeros((L,), jnp.float32)
        ones = jnp.ones((L,), jnp.float32)
        zi = jnp.zeros((L,), jnp.int32)

        def z64(i, _):
            for t in range(d // L):
                buf0[i, pl.ds(t * L, L)] = zeros
            return 0

        lax.fori_loop(0, 64, z64, 0)
        for t in range(G // L):
            degones[pl.ds(t * L, L)] = ones
        for t in range(64 // L):
            zdeg[pl.ds(t * L, L)] = zeros
        nzch = pad // 64

        def one_pass(p):
            lo = cid * half + p * quarter

            def zc(kk, _):
                i = sid + 16 * kk

                @pl.when(i < nzch)
                def _():
                    pltpu.sync_copy(buf0.at[pl.ds(0, 64)],
                                    acc_sh.at[pl.ds(i * 64, 64)])
                    pltpu.sync_copy(zdeg, deg_sh.at[pl.ds(i * 64, 64)])
                return 0

            lax.fori_loop(0, (nzch + 15) // 16, zc, 0)
            plsc.subcore_barrier()

            def mb(kk, _):
                base = kk * G
                for t in range(G // L):
                    addidx[pl.ds(t * L, L)] = cdst[pl.ds(base + t * L, L)]
                cpg = pltpu.async_copy(tbl_hbm.at[csrc.at[pl.ds(base, G)]],
                                       buf0.at[pl.ds(0, G)], sem0)
                cpg.wait()
                pltpu.sync_copy(buf0.at[pl.ds(0, G)], acc_sh.at[addidx], add=True)
                pltpu.sync_copy(degones, deg_sh.at[addidx], add=True)
                return 0

            def cb(i, off):
                sv = srcv[pl.ds(i * L, L)]
                dv = dstv[pl.ds(i * L, L)]
                msk = (dv >= lo) & (dv < lo + quarter)
                plsc.store_compressed(csrc.at[pl.ds(off, L)], sv, mask=msk)
                plsc.store_compressed(cdst.at[pl.ds(off, L)], dv - lo, mask=msk)
                pc = plsc.all_reduce_population_count(msk)
                return off + pc[0]

            off = 0
            for sg in range(nseg):
                pltpu.sync_copy(src_hbm.at[pl.ds(sid * ept + sg * seg, seg)], srcv)
                pltpu.sync_copy(dst_hbm.at[pl.ds(sid * ept + sg * seg, seg)], dstv)
                off = lax.fori_loop(0, seg // L, cb, off)
                nfull = off // G
                lax.fori_loop(0, nfull, mb, 0)
                for t in range(G // L):
                    csrc[pl.ds(t * L, L)] = csrc[pl.ds(nfull * G + t * L, L)]
                    cdst[pl.ds(t * L, L)] = cdst[pl.ds(nfull * G + t * L, L)]
                off = off - nfull * G
            for t in range(G // L):
                csrc[pl.ds(off + t * L, L)] = zi
                cdst[pl.ds(off + t * L, L)] = zi + quarter
            lax.fori_loop(0, (off + G - 1) // G, mb, 0)
            plsc.subcore_barrier()

            def rb(r, _):
                den = jnp.maximum(degbuf[pl.ds(r, L)][0], 1.0)
                if with_s:
                    for t in range(d // (2 * L)):
                        v = buf0[r, pl.ds(t * L, L)] / den
                        v = v + sbuf[r, pl.ds(d // 2 + t * L, L)]
                        buf0[r, pl.ds(t * L, L)] = v
                else:
                    for t in range(d // L):
                        v = buf0[r, pl.ds(t * L, L)] / den
                        buf0[r, pl.ds(t * L, L)] = v
                return 0

            for kk in range((nech + 15) // 16):
                i = sid + 16 * kk

                @pl.when(i < nech)
                def _(i=i):
                    r0 = i * R
                    pltpu.sync_copy(acc_sh.at[pl.ds(r0, R)],
                                    buf0.at[pl.ds(0, R)])
                    pltpu.sync_copy(deg_sh.at[pl.ds(r0, R)],
                                    degbuf.at[pl.ds(0, R)])
                    if with_s:
                        pltpu.sync_copy(tbl_hbm.at[pl.ds(lo + r0, R)],
                                        sbuf.at[pl.ds(0, R)])
                    lax.fori_loop(0, R, rb, 0)
                    pltpu.sync_copy(buf0.at[pl.ds(0, R)],
                                    out_hbm.at[pl.ds(lo + r0, R)])

        for p in range(npass):
            if p:
                plsc.subcore_barrier()
                lax.fori_loop(0, 64, z64, 0)
            one_pass(p)

    return functools.partial(
        pl.kernel,
        mesh=mesh,
        compiler_params=pltpu.CompilerParams(needs_layout_passes=False),
        out_type=jax.ShapeDtypeStruct((n_dst, d), jnp.float32),
        scratch_types=scratch,
    )(body)


_pull0 = _make_pull(50000 - BS, 128)
_pull1 = _make_pull(N_DST0 - BS, 128)
_conv0 = _make_conv(E0, N_DST0, 128, False, 40, 2)
_conv1 = _make_conv(E1, N_DST1, 128, True, 40, 1)


def _mm1(x_full, hn, ws, wn, b):
    blk = 1000

    def body(xd_ref, hn_ref, ws_ref, wn_ref, b_ref, o_ref):
        acc = jnp.dot(xd_ref[...], ws_ref[...], preferred_element_type=jnp.float32)
        acc += jnp.dot(hn_ref[...], wn_ref[...], preferred_element_type=jnp.float32)
        o_ref[...] = jnp.maximum(acc + b_ref[...], 0.0)

    return pl.pallas_call(
        body,
        grid=(N_DST0 // blk,),
        in_specs=[
            pl.BlockSpec((blk, 128), lambda i: (i, 0)),
            pl.BlockSpec((blk, 128), lambda i: (i, 0)),
            pl.BlockSpec((128, 128), lambda i: (0, 0)),
            pl.BlockSpec((128, 128), lambda i: (0, 0)),
            pl.BlockSpec((1, 128), lambda i: (0, 0)),
        ],
        out_specs=pl.BlockSpec((blk, 128), lambda i: (i, 0)),
        out_shape=jax.ShapeDtypeStruct((N_DST0, 128), jnp.float32),
    )(x_full, hn, ws, wn, b.reshape(1, 128))


def _mm2(h_full, wn, ws, b):
    blk = 1000
    wcat = jnp.concatenate([wn, ws], axis=1)
    bcat = jnp.concatenate([jnp.zeros((64,), jnp.float32), b]).reshape(1, 128)

    def body(hf_ref, w_ref, b_ref, p_ref):
        p_ref[...] = jnp.dot(hf_ref[...], w_ref[...],
                             preferred_element_type=jnp.float32) + b_ref[...]

    return pl.pallas_call(
        body,
        grid=(N_DST0 // blk,),
        in_specs=[
            pl.BlockSpec((blk, 128), lambda i: (i, 0)),
            pl.BlockSpec((128, 128), lambda i: (0, 0)),
            pl.BlockSpec((1, 128), lambda i: (0, 0)),
        ],
        out_specs=pl.BlockSpec((blk, 128), lambda i: (i, 0)),
        out_shape=jax.ShapeDtypeStruct((N_DST0, 128), jnp.float32),
    )(h_full, wcat, bcat)


def kernel(x, src0, dst0, src1, dst1, n_id0, n_id1, batch_size, hist0, hist1,
           W_self1, W_neigh1, b1, W_self2, W_neigh2, b2):
    del batch_size
    x_full = _pull0(x, n_id0, hist0)
    hn0 = _conv0(x_full, src0, dst0)
    h = _mm1(x_full, hn0, W_self1, W_neigh1, b1)
    h_full = _pull1(h, n_id1, hist1)
    pcat = _mm2(h_full, W_neigh2, W_self2, b2)
    return _conv1(pcat, src1, dst1)[:, :64]
